# baseline (device time: 145151 ns/iter reference)
import jax
import jax.numpy as jnp
from jax import lax
from jax.experimental import pallas as pl
from jax.experimental.pallas import tpu as pltpu

N_DEV = 8
SQ = 1024
SKV = 1024
H_LOC = 8
DH = 128
D_LOC = H_LOC * DH
BLK = 64
SCALE = 0.08838834764831843


def kernel(x, Wq, K_ext, V_ext, Wo):
    my = lax.axis_index("i")
    Wq_loc = lax.dynamic_slice(Wq, (0, my * D_LOC), (Wq.shape[0], D_LOC))
    Wo_loc = lax.dynamic_slice(Wo, (my * D_LOC, 0), (D_LOC, Wo.shape[1]))

    def body(x_ref, wq_ref, k_ref, v_ref, wo_ref, out_ref,
             comm_ref, ctx_ref, qc_ref, rs_send, rs_recv, ag_send, ag_recv):
        my_pos = lax.axis_index("i")
        left = lax.rem(my_pos - 1 + N_DEV, N_DEV)
        right = lax.rem(my_pos + 1, N_DEV)

        barrier_sem = pltpu.get_barrier_semaphore()
        for nbr in (left, right):
            pl.semaphore_signal(barrier_sem, inc=1, device_id=(nbr,),
                                device_id_type=pl.DeviceIdType.MESH)
        pl.semaphore_wait(barrier_sem, 2)

        R = SQ // N_DEV

        def compute_chunk(c):
            xr = x_ref[0, pl.ds(c * R, R), :]
            qc_ref[...] = jnp.dot(xr, wq_ref[...],
                                  preferred_element_type=jnp.float32)
            rb = 2 * c + lax.broadcasted_iota(jnp.int32, (R, SKV), 0) // BLK
            cb = lax.broadcasted_iota(jnp.int32, (R, SKV), 1) // BLK
            mask = cb <= rb

            def head_body(h, _):
                q = qc_ref[:, pl.ds(h * DH, DH)]
                k = jnp.reshape(k_ref[0, :, pl.ds(h, 1), :], (SKV, DH))
                v = jnp.reshape(v_ref[0, :, pl.ds(h, 1), :], (SKV, DH))
                s = lax.dot_general(q, k, (((1,), (1,)), ((), ())),
                                    preferred_element_type=jnp.float32) * SCALE
                s = jnp.where(mask, s, -1e9)
                m = jnp.max(s, axis=-1, keepdims=True)
                w = jnp.exp(s - m)
                w = w / jnp.sum(w, axis=-1, keepdims=True)
                ctx_ref[:, pl.ds(h * DH, DH)] = jnp.dot(
                    w, v, preferred_element_type=jnp.float32)
                return 0

            lax.fori_loop(0, H_LOC, head_body, 0)
            out_ref[0, pl.ds(c * R, R), :] = jnp.dot(
                ctx_ref[...], wo_ref[...], preferred_element_type=jnp.float32)

        def compute_chunk_pair(ca, cb2):
            def pair_body(i, _):
                compute_chunk(jnp.where(i == 0, ca, cb2))
                return 0
            lax.fori_loop(0, 2, pair_body, 0)

        compute_chunk(my_pos)

        HALF = 512

        def rows(c):
            return pl.ds(c * R, R)

        def cols(dirn):
            return pl.ds(dirn * HALF, HALF)

        for h in range(N_DEV - 1):
            rdmas = []
            for dirn in range(2):
                nbr = right if dirn == 0 else left
                if h == 0:
                    src = out_ref.at[0, rows(my_pos), cols(dirn)]
                else:
                    src = comm_ref.at[dirn, h % 2]
                rdma = pltpu.make_async_remote_copy(
                    src_ref=src,
                    dst_ref=comm_ref.at[dirn, (h + 1) % 2],
                    send_sem=rs_send.at[dirn, h],
                    recv_sem=rs_recv.at[dirn, h],
                    device_id=(nbr,),
                    device_id_type=pl.DeviceIdType.MESH,
                )
                rdma.start()
                rdmas.append(rdma)
            if h <= 2:
                compute_chunk_pair(lax.rem(my_pos - h - 1 + N_DEV, N_DEV),
                                   lax.rem(my_pos + h + 1, N_DEV))
            elif h == 3:
                compute_chunk(lax.rem(my_pos + 4, N_DEV))
            for dirn in range(2):
                rdmas[dirn].wait()
                if dirn == 0:
                    c = lax.rem(my_pos - h - 1 + N_DEV, N_DEV)
                else:
                    c = lax.rem(my_pos + h + 1, N_DEV)
                acc = comm_ref[dirn, (h + 1) % 2] + out_ref[0, rows(c), cols(dirn)]
                if h < N_DEV - 2:
                    comm_ref[dirn, (h + 1) % 2] = acc
                else:
                    out_ref[0, rows(c), cols(dirn)] = acc

        for k in range(N_DEV - 1):
            rdmas = []
            for dirn in range(2):
                nbr = right if dirn == 0 else left
                if dirn == 0:
                    c_send = lax.rem(my_pos + 1 - k + N_DEV, N_DEV)
                else:
                    c_send = lax.rem(my_pos - 1 + k + N_DEV, N_DEV)
                rdma = pltpu.make_async_remote_copy(
                    src_ref=out_ref.at[0, rows(c_send), cols(dirn)],
                    dst_ref=out_ref.at[0, rows(c_send), cols(dirn)],
                    send_sem=ag_send.at[dirn, k],
                    recv_sem=ag_recv.at[dirn, k],
                    device_id=(nbr,),
                    device_id_type=pl.DeviceIdType.MESH,
                )
                rdma.start()
                rdmas.append(rdma)
            for dirn in range(2):
                rdmas[dirn].wait()

    return pl.pallas_call(
        body,
        out_shape=jax.ShapeDtypeStruct((1, SQ, Wo.shape[1]), jnp.float32),
        in_specs=[pl.BlockSpec(memory_space=pltpu.VMEM)] * 5,
        out_specs=pl.BlockSpec(memory_space=pltpu.VMEM),
        scratch_shapes=[
            pltpu.VMEM((2, 2, SQ // N_DEV, 512), jnp.float32),
            pltpu.VMEM((SQ // N_DEV, 1024), jnp.float32),
            pltpu.VMEM((SQ // N_DEV, 1024), jnp.float32),
            pltpu.SemaphoreType.DMA((2, N_DEV - 1)),
            pltpu.SemaphoreType.DMA((2, N_DEV - 1)),
            pltpu.SemaphoreType.DMA((2, N_DEV - 1)),
            pltpu.SemaphoreType.DMA((2, N_DEV - 1)),
        ],
        compiler_params=pltpu.CompilerParams(collective_id=0),
    )(x, Wq_loc, K_ext, V_ext, Wo_loc)


# device time: 78691 ns/iter; 1.8446x vs baseline; 1.8446x over previous
import jax
import jax.numpy as jnp
from jax import lax
from jax.experimental import pallas as pl
from jax.experimental.pallas import tpu as pltpu

N_DEV = 8
SQ = 1024
SKV = 1024
H_LOC = 8
DH = 128
D_LOC = H_LOC * DH
BLK = 64
SCALE = 0.08838834764831843

ORDERS = [[1, 3, 4], [3, 4, 1], [4, 1, 3]]
COLS = [(0, 384), (384, 384), (768, 256)]


def kernel(x, Wq, K_ext, V_ext, Wo):
    my = lax.axis_index("i")
    Wq_loc = lax.dynamic_slice(Wq, (0, my * D_LOC), (Wq.shape[0], D_LOC))
    Wo_loc = lax.dynamic_slice(Wo, (my * D_LOC, 0), (D_LOC, Wo.shape[1]))

    def body(x_ref, wq_ref, k_ref, v_ref, wo_ref, out_ref,
             comm_ref, rs_send, rs_recv, ag_send, ag_recv):
        my_pos = lax.axis_index("i")
        b0 = my_pos % 2
        b1 = (my_pos // 2) % 2
        b2 = my_pos // 4
        dual = {1: b0 ^ b1, 3: b1, 4: b2}

        barrier_sem = pltpu.get_barrier_semaphore()
        for m in (1, 3, 4):
            pl.semaphore_signal(barrier_sem, inc=1,
                                device_id=(my_pos ^ m,),
                                device_id_type=pl.DeviceIdType.MESH)
        pl.semaphore_wait(barrier_sem, 3)

        xm = x_ref[0]
        Q = jnp.dot(xm, wq_ref[...], preferred_element_type=jnp.float32)

        rb = lax.broadcasted_iota(jnp.int32, (SQ, SKV), 0) // BLK
        cb = lax.broadcasted_iota(jnp.int32, (SQ, SKV), 1) // BLK
        mask = cb <= rb

        ctx_parts = []
        for h in range(H_LOC):
            q = Q[:, h * DH:(h + 1) * DH]
            k = k_ref[0, :, h, :]
            v = v_ref[0, :, h, :]
            s = lax.dot_general(q, k, (((1,), (1,)), ((), ())),
                                preferred_element_type=jnp.float32) * SCALE
            s = jnp.where(mask, s, -1e9)
            mx = jnp.max(s, axis=-1, keepdims=True)
            w = jnp.exp(s - mx)
            w = w / jnp.sum(w, axis=-1, keepdims=True)
            ctx_parts.append(jnp.dot(w, v, preferred_element_type=jnp.float32))
        ctx = jnp.concatenate(ctx_parts, axis=1)
        out_ref[0] = jnp.dot(ctx, wo_ref[...],
                             preferred_element_type=jnp.float32)

        def vbits(p):
            return [dual[ORDERS[p][j]] for j in range(3)]

        for lvl in range(3):
            half = 512 >> lvl
            rdmas = []
            for p in range(3):
                vb = vbits(p)
                partner = my_pos ^ ORDERS[p][lvl]
                base = sum((vb[j] * (512 >> j) for j in range(lvl)), 0)
                s_start = base + (1 - vb[lvl]) * half
                c0, cw = COLS[p]
                rdma = pltpu.make_async_remote_copy(
                    src_ref=out_ref.at[0, pl.ds(s_start, half), pl.ds(c0, cw)],
                    dst_ref=comm_ref.at[lvl, pl.ds(0, half), pl.ds(c0, cw)],
                    send_sem=rs_send.at[p, lvl],
                    recv_sem=rs_recv.at[p, lvl],
                    device_id=(partner,),
                    device_id_type=pl.DeviceIdType.MESH,
                )
                rdma.start()
                rdmas.append(rdma)
            for p in range(3):
                rdmas[p].wait()
                vb = vbits(p)
                base = sum((vb[j] * (512 >> j) for j in range(lvl)), 0)
                k_start = base + vb[lvl] * half
                c0, cw = COLS[p]
                out_ref[0, pl.ds(k_start, half), pl.ds(c0, cw)] = (
                    out_ref[0, pl.ds(k_start, half), pl.ds(c0, cw)]
                    + comm_ref[lvl, pl.ds(0, half), pl.ds(c0, cw)]
                )

        for lvl in range(3):
            g = 128 << lvl
            rs_lvl = 2 - lvl
            rdmas = []
            for p in range(3):
                vb = vbits(p)
                partner = my_pos ^ ORDERS[p][rs_lvl]
                start = sum((vb[j] * (512 >> j) for j in range(rs_lvl + 1)), 0)
                c0, cw = COLS[p]
                rdma = pltpu.make_async_remote_copy(
                    src_ref=out_ref.at[0, pl.ds(start, g), pl.ds(c0, cw)],
                    dst_ref=out_ref.at[0, pl.ds(start, g), pl.ds(c0, cw)],
                    send_sem=ag_send.at[p, lvl],
                    recv_sem=ag_recv.at[p, lvl],
                    device_id=(partner,),
                    device_id_type=pl.DeviceIdType.MESH,
                )
                rdma.start()
                rdmas.append(rdma)
            for p in range(3):
                rdmas[p].wait()

    return pl.pallas_call(
        body,
        out_shape=jax.ShapeDtypeStruct((1, SQ, Wo.shape[1]), jnp.float32),
        in_specs=[pl.BlockSpec(memory_space=pltpu.VMEM)] * 5,
        out_specs=pl.BlockSpec(memory_space=pltpu.VMEM),
        scratch_shapes=[
            pltpu.VMEM((3, 512, 1024), jnp.float32),
            pltpu.SemaphoreType.DMA((3, 3)),
            pltpu.SemaphoreType.DMA((3, 3)),
            pltpu.SemaphoreType.DMA((3, 3)),
            pltpu.SemaphoreType.DMA((3, 3)),
        ],
        compiler_params=pltpu.CompilerParams(collective_id=0),
    )(x, Wq_loc, K_ext, V_ext, Wo_loc)


# device time: 77540 ns/iter; 1.8719x vs baseline; 1.0148x over previous
import jax
import jax.numpy as jnp
from jax import lax
from jax.experimental import pallas as pl
from jax.experimental.pallas import tpu as pltpu

N_DEV = 8
SQ = 1024
SKV = 1024
H_LOC = 8
DH = 128
D_LOC = H_LOC * DH
BLK = 64
SCALE = 0.08838834764831843

ORDERS = [[1, 3, 4], [3, 4, 1], [4, 1, 3]]
COLS = [(0, 384), (384, 384), (768, 256)]


def kernel(x, Wq, K_ext, V_ext, Wo):
    my = lax.axis_index("i")
    Wq_loc = lax.dynamic_slice(Wq, (0, my * D_LOC), (Wq.shape[0], D_LOC))
    Wo_loc = lax.dynamic_slice(Wo, (my * D_LOC, 0), (D_LOC, Wo.shape[1]))

    def body(x_ref, wq_ref, k_ref, v_ref, wo_ref, out_ref,
             comm_ref, ctx_ref, rs_send, rs_recv, ag_send, ag_recv):
        my_pos = lax.axis_index("i")
        b0 = my_pos % 2
        b1 = (my_pos // 2) % 2
        b2 = my_pos // 4
        dual = {1: b0 ^ b1, 3: b1, 4: b2}

        barrier_sem = pltpu.get_barrier_semaphore()
        for m in (1, 3, 4):
            pl.semaphore_signal(barrier_sem, inc=1,
                                device_id=(my_pos ^ m,),
                                device_id_type=pl.DeviceIdType.MESH)
        pl.semaphore_wait(barrier_sem, 3)

        xm = x_ref[0]
        Q = jnp.dot(xm, wq_ref[...], preferred_element_type=jnp.float32)

        HR = SQ // 2
        rbT = lax.broadcasted_iota(jnp.int32, (HR, HR), 0) // BLK
        cbT = lax.broadcasted_iota(jnp.int32, (HR, HR), 1) // BLK
        maskT = cbT <= rbT
        rbB = HR // BLK + lax.broadcasted_iota(jnp.int32, (HR, SKV), 0) // BLK
        cbB = lax.broadcasted_iota(jnp.int32, (HR, SKV), 1) // BLK
        maskB = cbB <= rbB

        for h in range(H_LOC):
            k = k_ref[0, :, h, :]
            v = v_ref[0, :, h, :]
            for mask, r0, kl in ((maskT, 0, HR), (maskB, HR, SKV)):
                q = Q[r0:r0 + HR, h * DH:(h + 1) * DH]
                s = lax.dot_general(q, k[:kl, :], (((1,), (1,)), ((), ())),
                                    preferred_element_type=jnp.float32) * SCALE
                s = jnp.where(mask, s, -1e9)
                mx = jnp.max(s, axis=-1, keepdims=True)
                w = jnp.exp(s - mx)
                w = w / jnp.sum(w, axis=-1, keepdims=True)
                ctx_ref[r0:r0 + HR, h * DH:(h + 1) * DH] = jnp.dot(
                    w, v[:kl, :], preferred_element_type=jnp.float32)

        def vbits(p):
            return [dual[ORDERS[p][j]] for j in range(3)]

        for p in range(3):
            vb = vbits(p)
            c0, cw = COLS[p]
            s_start = (1 - vb[0]) * HR
            out_ref[0, pl.ds(s_start, HR), pl.ds(c0, cw)] = jnp.dot(
                ctx_ref[pl.ds(s_start, HR), :], wo_ref[:, c0:c0 + cw],
                preferred_element_type=jnp.float32)

        for lvl in range(3):
            half = 512 >> lvl
            rdmas = []
            for p in range(3):
                vb = vbits(p)
                partner = my_pos ^ ORDERS[p][lvl]
                base = sum((vb[j] * (512 >> j) for j in range(lvl)), 0)
                s_start = base + (1 - vb[lvl]) * half
                c0, cw = COLS[p]
                rdma = pltpu.make_async_remote_copy(
                    src_ref=out_ref.at[0, pl.ds(s_start, half), pl.ds(c0, cw)],
                    dst_ref=comm_ref.at[lvl, pl.ds(0, half), pl.ds(c0, cw)],
                    send_sem=rs_send.at[p, lvl],
                    recv_sem=rs_recv.at[p, lvl],
                    device_id=(partner,),
                    device_id_type=pl.DeviceIdType.MESH,
                )
                rdma.start()
                rdmas.append(rdma)
            if lvl == 0:
                for p in range(3):
                    vb = vbits(p)
                    c0, cw = COLS[p]
                    k_start = vb[0] * HR
                    out_ref[0, pl.ds(k_start, HR), pl.ds(c0, cw)] = jnp.dot(
                        ctx_ref[pl.ds(k_start, HR), :], wo_ref[:, c0:c0 + cw],
                        preferred_element_type=jnp.float32)
            for p in range(3):
                rdmas[p].wait()
                vb = vbits(p)
                base = sum((vb[j] * (512 >> j) for j in range(lvl)), 0)
                k_start = base + vb[lvl] * half
                c0, cw = COLS[p]
                out_ref[0, pl.ds(k_start, half), pl.ds(c0, cw)] = (
                    out_ref[0, pl.ds(k_start, half), pl.ds(c0, cw)]
                    + comm_ref[lvl, pl.ds(0, half), pl.ds(c0, cw)]
                )

        for lvl in range(3):
            g = 128 << lvl
            rs_lvl = 2 - lvl
            rdmas = []
            for p in range(3):
                vb = vbits(p)
                partner = my_pos ^ ORDERS[p][rs_lvl]
                start = sum((vb[j] * (512 >> j) for j in range(rs_lvl + 1)), 0)
                c0, cw = COLS[p]
                rdma = pltpu.make_async_remote_copy(
                    src_ref=out_ref.at[0, pl.ds(start, g), pl.ds(c0, cw)],
                    dst_ref=out_ref.at[0, pl.ds(start, g), pl.ds(c0, cw)],
                    send_sem=ag_send.at[p, lvl],
                    recv_sem=ag_recv.at[p, lvl],
                    device_id=(partner,),
                    device_id_type=pl.DeviceIdType.MESH,
                )
                rdma.start()
                rdmas.append(rdma)
            for p in range(3):
                rdmas[p].wait()

    return pl.pallas_call(
        body,
        out_shape=jax.ShapeDtypeStruct((1, SQ, Wo.shape[1]), jnp.float32),
        in_specs=[pl.BlockSpec(memory_space=pltpu.VMEM)] * 5,
        out_specs=pl.BlockSpec(memory_space=pltpu.VMEM),
        scratch_shapes=[
            pltpu.VMEM((3, 512, 1024), jnp.float32),
            pltpu.VMEM((SQ, 1024), jnp.float32),
            pltpu.SemaphoreType.DMA((3, 3)),
            pltpu.SemaphoreType.DMA((3, 3)),
            pltpu.SemaphoreType.DMA((3, 3)),
            pltpu.SemaphoreType.DMA((3, 3)),
        ],
        compiler_params=pltpu.CompilerParams(collective_id=0),
    )(x, Wq_loc, K_ext, V_ext, Wo_loc)
